# baseline (speedup 1.0000x reference)
import dataclasses
import functools

import numpy as np

import jax
import jax.numpy as jnp
from jax import lax
from jax.experimental import pallas as pl
from jax.experimental.pallas import tpu as pltpu
from jax.experimental.pallas import tpu_sc as plsc

NC = 2
NS = 16
NW = NC * NS
BLK = 128


def _round_up(a, b):
    return (a + b - 1) // b * b


def _sc_segsum(g, src3, dst3, zeros_blk, n_pad):
    dim = g.shape[1]
    kpt = src3.shape[1]
    kh = kpt // 2
    rows_per_tile = n_pad // NS
    mesh = plsc.VectorSubcoreMesh(core_axis_name="c", subcore_axis_name="s")

    @functools.partial(
        pl.kernel,
        out_type=jax.ShapeDtypeStruct((NC, n_pad, dim), jnp.float32),
        mesh=mesh,
        scratch_types=[
            pltpu.VMEM((kh, BLK), jnp.int32),
            pltpu.VMEM((kh, BLK), jnp.int32),
            pltpu.VMEM((BLK, dim), jnp.float32),
            pltpu.VMEM((BLK, dim), jnp.float32),
            pltpu.VMEM_SHARED((n_pad, dim), jnp.float32),
            pltpu.SemaphoreType.DMA,
            pltpu.SemaphoreType.DMA,
            pltpu.SemaphoreType.DMA,
            pltpu.SemaphoreType.DMA,
        ],
    )
    def seg_kernel(g_hbm, src_hbm, dst_hbm, zeros_hbm, out_hbm,
                   src_v, dst_v, rows_a, rows_b, acc_sh,
                   gsem_a, gsem_b, ssem_a, ssem_b):
        c = lax.axis_index("c")
        s = lax.axis_index("s")
        wid = c * NS + s
        r0 = s * rows_per_tile
        pltpu.sync_copy(zeros_hbm, rows_b)

        @pl.loop(0, rows_per_tile, step=BLK)
        def _(r):
            pltpu.async_copy(rows_b, acc_sh.at[pl.ds(r0 + r, BLK)], ssem_a)

        def run_half(half):
            pltpu.sync_copy(src_hbm.at[wid, pl.ds(half * kh, kh)], src_v)
            pltpu.sync_copy(dst_hbm.at[wid, pl.ds(half * kh, kh)], dst_v)

            pltpu.async_copy(g_hbm.at[src_v.at[0]], rows_a, gsem_a)

            if half == 0:
                @pl.loop(0, rows_per_tile, step=BLK)
                def _(r):
                    pltpu.make_async_copy(
                        rows_b, acc_sh.at[pl.ds(r0 + r, BLK)], ssem_a).wait()

                plsc.subcore_barrier()

            def pair(k, last):
                pltpu.async_copy(g_hbm.at[src_v.at[k + 1]], rows_b, gsem_b)
                pltpu.make_async_copy(g_hbm.at[src_v.at[k]], rows_a, gsem_a).wait()
                pltpu.async_copy(rows_a, acc_sh.at[dst_v.at[k]], ssem_a, add=True)
                pltpu.make_async_copy(g_hbm.at[src_v.at[k + 1]], rows_b, gsem_b).wait()
                pltpu.async_copy(rows_b, acc_sh.at[dst_v.at[k + 1]], ssem_b, add=True)
                pltpu.make_async_copy(rows_a, acc_sh.at[dst_v.at[k]], ssem_a).wait()

                if last is None:
                    pltpu.async_copy(g_hbm.at[src_v.at[k + 2]], rows_a, gsem_a)
                else:
                    @pl.when(last)
                    def _():
                        pltpu.async_copy(g_hbm.at[src_v.at[k + 2]], rows_a, gsem_a)

                pltpu.make_async_copy(rows_b, acc_sh.at[dst_v.at[k + 1]], ssem_b).wait()

            @pl.loop(0, kh // 4)
            def _(q):
                k = 4 * q
                pair(k, None)
                pair(k + 2, k + 4 < kh)

        run_half(0)
        run_half(1)
        plsc.subcore_barrier()
        pltpu.sync_copy(acc_sh.at[pl.ds(r0, rows_per_tile)],
                        out_hbm.at[c, pl.ds(r0, rows_per_tile)])

    return seg_kernel(g, src3, dst3, zeros_blk)


def _sc_degree(dst3, n_pad):
    kpt = dst3.shape[1]
    mesh = plsc.VectorSubcoreMesh(core_axis_name="c", subcore_axis_name="s")
    cp = pltpu.CompilerParams()
    if "needs_layout_passes" in pltpu.CompilerParams.__dataclass_fields__:
        cp = dataclasses.replace(cp, needs_layout_passes=False)

    @functools.partial(
        pl.kernel,
        out_type=jax.ShapeDtypeStruct((NW, n_pad), jnp.float32),
        mesh=mesh,
        compiler_params=cp,
        scratch_types=[
            pltpu.VMEM((kpt, BLK), jnp.int32),
            pltpu.VMEM((n_pad,), jnp.float32),
        ],
    )
    def deg_kernel(dst_hbm, out_hbm, dst_v, hist):
        c = lax.axis_index("c")
        s = lax.axis_index("s")
        wid = c * NS + s
        pltpu.sync_copy(dst_hbm.at[wid], dst_v)

        @pl.loop(0, n_pad, step=16)
        def _(i):
            hist[pl.ds(i, 16)] = jnp.zeros((16,), jnp.float32)

        ones = jnp.ones((16,), jnp.float32)

        @pl.loop(0, kpt)
        def _(j):
            @pl.loop(0, BLK, step=16)
            def _(k):
                idx = dst_v[j, pl.ds(k, 16)]
                plsc.addupdate_scatter(hist, [idx], ones)

        pltpu.sync_copy(hist, out_hbm.at[wid])

    return deg_kernel(dst3)


def _mm_t(a, w):
    return lax.dot_general(a, w, (((1,), (1,)), ((), ())),
                           preferred_element_type=jnp.float32,
                           precision=lax.Precision.HIGHEST)


def _tc_lin_relu(x, W1, b1):
    def body(x_ref, w_ref, b_ref, o_ref):
        o_ref[...] = jnp.maximum(_mm_t(x_ref[...], w_ref[...]) + b_ref[...], 0.0)

    return pl.pallas_call(
        body,
        out_shape=jax.ShapeDtypeStruct((x.shape[0], W1.shape[0]), jnp.float32),
    )(x, W1, b1.reshape(1, -1))


def _tc_conv_pre(h, Wc, degp):
    n = h.shape[0]

    def body(h_ref, w_ref, d_ref, hp_ref, g_ref, dis_ref):
        hp = _mm_t(h_ref[...], w_ref[...])
        ones_nw = jnp.ones((NW, 1), jnp.float32)
        deg = lax.dot_general(d_ref[:, :n], ones_nw, (((0,), (0,)), ((), ())),
                              preferred_element_type=jnp.float32,
                              precision=lax.Precision.DEFAULT) + 1.0
        dis = lax.rsqrt(deg)
        hp_ref[...] = hp
        g_ref[...] = hp * dis
        dis_ref[...] = dis

    return pl.pallas_call(
        body,
        out_shape=(
            jax.ShapeDtypeStruct((n, Wc.shape[0]), jnp.float32),
            jax.ShapeDtypeStruct((n, Wc.shape[0]), jnp.float32),
            jax.ShapeDtypeStruct((n, 1), jnp.float32),
        ),
    )(h, Wc, degp)


def _tc_conv_post_pre(S, hp, dis, bc, Wc2):
    n = hp.shape[0]

    def body(s_ref, hp_ref, dis_ref, b_ref, w_ref, hp2_ref, g2_ref):
        dis = dis_ref[...]
        ssum = s_ref[0, :n] + s_ref[1, :n]
        agg = dis * ssum + dis * dis * hp_ref[...] + b_ref[...]
        x2 = jnp.maximum(agg, 0.0)
        hp2 = _mm_t(x2, w_ref[...])
        hp2_ref[...] = hp2
        g2_ref[...] = hp2 * dis

    return pl.pallas_call(
        body,
        out_shape=(
            jax.ShapeDtypeStruct((n, Wc2.shape[0]), jnp.float32),
            jax.ShapeDtypeStruct((n, Wc2.shape[0]), jnp.float32),
        ),
    )(S, hp, dis, bc.reshape(1, -1), Wc2)


def _tc_final(S, hp, dis, bc):
    n, dim = hp.shape

    def body(s_ref, hp_ref, dis_ref, b_ref, o_ref):
        dis = dis_ref[...]
        ssum = s_ref[0, :n] + s_ref[1, :n]
        z = dis * ssum + dis * dis * hp_ref[...] + b_ref[...]
        m = jnp.max(z, axis=1, keepdims=True)
        zs = z - m
        lse = jnp.log(jnp.sum(jnp.exp(zs), axis=1, keepdims=True))
        o_ref[...] = zs - lse

    return pl.pallas_call(
        body,
        out_shape=jax.ShapeDtypeStruct((n, dim), jnp.float32),
    )(S, hp, dis, bc.reshape(1, -1))


def kernel(x, edge_index, W1, b1, Wc1, bc1, Wc2, bc2):
    n, _ = x.shape
    e = edge_index.shape[1]
    dh = W1.shape[0]

    n_pad = _round_up(n + 128, NS * BLK)
    kpt = _round_up(-(-e // (NW * BLK)), 8)
    e_pad = NW * kpt * BLK

    pad_iota = np.arange(e_pad - e, dtype=np.int32)
    pad_src = jnp.asarray(pad_iota % n)
    pad_dst = jnp.asarray(n + pad_iota % (n_pad - n))
    src = jnp.concatenate([edge_index[0], pad_src])
    dst = jnp.concatenate([edge_index[1], pad_dst])
    src3 = src.reshape(NW, kpt, BLK)
    dst3 = dst.reshape(NW, kpt, BLK)

    zeros_blk = jnp.zeros((BLK, dh), jnp.float32)

    degp = _sc_degree(dst3, n_pad)
    h1 = _tc_lin_relu(x, W1, b1)
    hp1, g1, dis = _tc_conv_pre(h1, Wc1, degp)
    S1 = _sc_segsum(g1, src3, dst3, zeros_blk, n_pad)
    hp2, g2 = _tc_conv_post_pre(S1, hp1, dis, bc1, Wc2)
    S2 = _sc_segsum(g2, src3, dst3, zeros_blk, n_pad)
    return _tc_final(S2, hp2, dis, bc2)


# speedup vs baseline: 1.0139x; 1.0139x over previous
import dataclasses
import functools

import numpy as np

import jax
import jax.numpy as jnp
from jax import lax
from jax.experimental import pallas as pl
from jax.experimental.pallas import tpu as pltpu
from jax.experimental.pallas import tpu_sc as plsc

NC = 2
NS = 16
NW = NC * NS
BLK = 128


def _round_up(a, b):
    return (a + b - 1) // b * b


def _sc_segsum(g, src3, dst3, zeros_blk, n_pad):
    dim = g.shape[1]
    kpt = src3.shape[1]
    kh = kpt // 2
    rows_per_tile = n_pad // NS
    mesh = plsc.VectorSubcoreMesh(core_axis_name="c", subcore_axis_name="s")

    @functools.partial(
        pl.kernel,
        out_type=jax.ShapeDtypeStruct((NC, n_pad, dim), jnp.float32),
        mesh=mesh,
        scratch_types=[
            pltpu.VMEM((kh, BLK), jnp.int32),
            pltpu.VMEM((kh, BLK), jnp.int32),
            pltpu.VMEM((BLK, dim), jnp.float32),
            pltpu.VMEM((BLK, dim), jnp.float32),
            pltpu.VMEM_SHARED((n_pad, dim), jnp.float32),
            pltpu.SemaphoreType.DMA,
            pltpu.SemaphoreType.DMA,
            pltpu.SemaphoreType.DMA,
            pltpu.SemaphoreType.DMA,
        ],
    )
    def seg_kernel(g_hbm, src_hbm, dst_hbm, zeros_hbm, out_hbm,
                   src_v, dst_v, rows_a, rows_b, acc_sh,
                   gsem_a, gsem_b, ssem_a, ssem_b):
        c = lax.axis_index("c")
        s = lax.axis_index("s")
        wid = c * NS + s
        r0 = s * rows_per_tile
        pltpu.sync_copy(zeros_hbm, rows_b)

        @pl.loop(0, rows_per_tile, step=BLK)
        def _(r):
            pltpu.async_copy(rows_b, acc_sh.at[pl.ds(r0 + r, BLK)], ssem_a)

        def run_half(half):
            pltpu.sync_copy(src_hbm.at[wid, pl.ds(half * kh, kh)], src_v)
            pltpu.sync_copy(dst_hbm.at[wid, pl.ds(half * kh, kh)], dst_v)

            pltpu.async_copy(g_hbm.at[src_v.at[0]], rows_a, gsem_a)

            if half == 0:
                @pl.loop(0, rows_per_tile, step=BLK)
                def _(r):
                    pltpu.make_async_copy(
                        rows_b, acc_sh.at[pl.ds(r0 + r, BLK)], ssem_a).wait()

                plsc.subcore_barrier()

            def pair(k, last):
                pltpu.async_copy(g_hbm.at[src_v.at[k + 1]], rows_b, gsem_b)
                pltpu.make_async_copy(g_hbm.at[src_v.at[k]], rows_a, gsem_a).wait()
                pltpu.async_copy(rows_a, acc_sh.at[dst_v.at[k]], ssem_a, add=True)
                pltpu.make_async_copy(g_hbm.at[src_v.at[k + 1]], rows_b, gsem_b).wait()
                pltpu.async_copy(rows_b, acc_sh.at[dst_v.at[k + 1]], ssem_b, add=True)
                pltpu.make_async_copy(rows_a, acc_sh.at[dst_v.at[k]], ssem_a).wait()

                if last is None:
                    pltpu.async_copy(g_hbm.at[src_v.at[k + 2]], rows_a, gsem_a)
                else:
                    @pl.when(last)
                    def _():
                        pltpu.async_copy(g_hbm.at[src_v.at[k + 2]], rows_a, gsem_a)

                pltpu.make_async_copy(rows_b, acc_sh.at[dst_v.at[k + 1]], ssem_b).wait()

            @pl.loop(0, kh // 4)
            def _(q):
                k = 4 * q
                pair(k, None)
                pair(k + 2, k + 4 < kh)

        run_half(0)
        run_half(1)
        plsc.subcore_barrier()
        pltpu.sync_copy(acc_sh.at[pl.ds(r0, rows_per_tile)],
                        out_hbm.at[c, pl.ds(r0, rows_per_tile)])

    return seg_kernel(g, src3, dst3, zeros_blk)


def _sc_degree(dst3, n_pad):
    kpt = dst3.shape[1]
    mesh = plsc.VectorSubcoreMesh(core_axis_name="c", subcore_axis_name="s")
    cp = pltpu.CompilerParams()
    if "needs_layout_passes" in pltpu.CompilerParams.__dataclass_fields__:
        cp = dataclasses.replace(cp, needs_layout_passes=False)

    @functools.partial(
        pl.kernel,
        out_type=jax.ShapeDtypeStruct((NW, n_pad), jnp.float32),
        mesh=mesh,
        compiler_params=cp,
        scratch_types=[
            pltpu.VMEM((kpt, BLK), jnp.int32),
            pltpu.VMEM((n_pad,), jnp.float32),
        ],
    )
    def deg_kernel(dst_hbm, out_hbm, dst_v, hist):
        c = lax.axis_index("c")
        s = lax.axis_index("s")
        wid = c * NS + s
        pltpu.sync_copy(dst_hbm.at[wid], dst_v)

        @pl.loop(0, n_pad, step=16)
        def _(i):
            hist[pl.ds(i, 16)] = jnp.zeros((16,), jnp.float32)

        ones = jnp.ones((16,), jnp.float32)

        @pl.loop(0, kpt)
        def _(j):
            @pl.loop(0, BLK, step=16)
            def _(k):
                idx = dst_v[j, pl.ds(k, 16)]
                plsc.addupdate_scatter(hist, [idx], ones)

        pltpu.sync_copy(hist, out_hbm.at[wid])

    return deg_kernel(dst3)


def _mm_t(a, w):
    return lax.dot_general(a, w, (((1,), (1,)), ((), ())),
                           preferred_element_type=jnp.float32,
                           precision=lax.Precision.DEFAULT)


def _tc_lin_relu(x, W1, b1):
    def body(x_ref, w_ref, b_ref, o_ref):
        o_ref[...] = jnp.maximum(_mm_t(x_ref[...], w_ref[...]) + b_ref[...], 0.0)

    return pl.pallas_call(
        body,
        out_shape=jax.ShapeDtypeStruct((x.shape[0], W1.shape[0]), jnp.float32),
    )(x, W1, b1.reshape(1, -1))


def _tc_conv_pre(h, Wc, degp):
    n = h.shape[0]

    def body(h_ref, w_ref, d_ref, hp_ref, g_ref, dis_ref):
        hp = _mm_t(h_ref[...], w_ref[...])
        ones_nw = jnp.ones((NW, 1), jnp.float32)
        deg = lax.dot_general(d_ref[:, :n], ones_nw, (((0,), (0,)), ((), ())),
                              preferred_element_type=jnp.float32,
                              precision=lax.Precision.DEFAULT) + 1.0
        dis = lax.rsqrt(deg)
        hp_ref[...] = hp
        g_ref[...] = hp * dis
        dis_ref[...] = dis

    return pl.pallas_call(
        body,
        out_shape=(
            jax.ShapeDtypeStruct((n, Wc.shape[0]), jnp.float32),
            jax.ShapeDtypeStruct((n, Wc.shape[0]), jnp.float32),
            jax.ShapeDtypeStruct((n, 1), jnp.float32),
        ),
    )(h, Wc, degp)


def _tc_conv_post_pre(S, hp, dis, bc, Wc2):
    n = hp.shape[0]

    def body(s_ref, hp_ref, dis_ref, b_ref, w_ref, hp2_ref, g2_ref):
        dis = dis_ref[...]
        ssum = s_ref[0, :n] + s_ref[1, :n]
        agg = dis * ssum + dis * dis * hp_ref[...] + b_ref[...]
        x2 = jnp.maximum(agg, 0.0)
        hp2 = _mm_t(x2, w_ref[...])
        hp2_ref[...] = hp2
        g2_ref[...] = hp2 * dis

    return pl.pallas_call(
        body,
        out_shape=(
            jax.ShapeDtypeStruct((n, Wc2.shape[0]), jnp.float32),
            jax.ShapeDtypeStruct((n, Wc2.shape[0]), jnp.float32),
        ),
    )(S, hp, dis, bc.reshape(1, -1), Wc2)


def _tc_final(S, hp, dis, bc):
    n, dim = hp.shape

    def body(s_ref, hp_ref, dis_ref, b_ref, o_ref):
        dis = dis_ref[...]
        ssum = s_ref[0, :n] + s_ref[1, :n]
        z = dis * ssum + dis * dis * hp_ref[...] + b_ref[...]
        m = jnp.max(z, axis=1, keepdims=True)
        zs = z - m
        lse = jnp.log(jnp.sum(jnp.exp(zs), axis=1, keepdims=True))
        o_ref[...] = zs - lse

    return pl.pallas_call(
        body,
        out_shape=jax.ShapeDtypeStruct((n, dim), jnp.float32),
    )(S, hp, dis, bc.reshape(1, -1))


def kernel(x, edge_index, W1, b1, Wc1, bc1, Wc2, bc2):
    n, _ = x.shape
    e = edge_index.shape[1]
    dh = W1.shape[0]

    n_pad = _round_up(n + 128, NS * BLK)
    kpt = _round_up(-(-e // (NW * BLK)), 8)
    e_pad = NW * kpt * BLK

    pad_iota = np.arange(e_pad - e, dtype=np.int32)
    pad_src = jnp.asarray(pad_iota % n)
    pad_dst = jnp.asarray(n + pad_iota % (n_pad - n))
    src = jnp.concatenate([edge_index[0], pad_src])
    dst = jnp.concatenate([edge_index[1], pad_dst])
    src3 = src.reshape(NW, kpt, BLK)
    dst3 = dst.reshape(NW, kpt, BLK)

    zeros_blk = jnp.zeros((BLK, dh), jnp.float32)

    degp = _sc_degree(dst3, n_pad)
    h1 = _tc_lin_relu(x, W1, b1)
    hp1, g1, dis = _tc_conv_pre(h1, Wc1, degp)
    S1 = _sc_segsum(g1, src3, dst3, zeros_blk, n_pad)
    hp2, g2 = _tc_conv_post_pre(S1, hp1, dis, bc1, Wc2)
    S2 = _sc_segsum(g2, src3, dst3, zeros_blk, n_pad)
    return _tc_final(S2, hp2, dis, bc2)


# speedup vs baseline: 1.0183x; 1.0044x over previous
import dataclasses
import functools

import numpy as np

import jax
import jax.numpy as jnp
from jax import lax
from jax.experimental import pallas as pl
from jax.experimental.pallas import tpu as pltpu
from jax.experimental.pallas import tpu_sc as plsc

NC = 2
NS = 16
NW = NC * NS
BLK = 128


def _round_up(a, b):
    return (a + b - 1) // b * b


def _sc_segsum(g, src3, dst3, zeros_blk, n_pad):
    dim = g.shape[1]
    kpt = src3.shape[1]
    kh = kpt // 2
    rows_per_tile = n_pad // NS
    mesh = plsc.VectorSubcoreMesh(core_axis_name="c", subcore_axis_name="s")

    @functools.partial(
        pl.kernel,
        out_type=jax.ShapeDtypeStruct((NC, n_pad, dim), jnp.float32),
        mesh=mesh,
        scratch_types=[
            pltpu.VMEM((kh, BLK), jnp.int32),
            pltpu.VMEM((kh, BLK), jnp.int32),
            pltpu.VMEM((BLK, dim), jnp.float32),
            pltpu.VMEM((BLK, dim), jnp.float32),
            pltpu.VMEM_SHARED((n_pad, dim), jnp.float32),
            pltpu.SemaphoreType.DMA,
            pltpu.SemaphoreType.DMA,
            pltpu.SemaphoreType.DMA,
            pltpu.SemaphoreType.DMA,
        ],
    )
    def seg_kernel(g_hbm, src_hbm, dst_hbm, zeros_hbm, out_hbm,
                   src_v, dst_v, rows_a, rows_b, acc_sh,
                   gsem_a, gsem_b, ssem_a, ssem_b):
        c = lax.axis_index("c")
        s = lax.axis_index("s")
        wid = c * NS + s
        r0 = s * rows_per_tile
        pltpu.sync_copy(zeros_hbm, rows_b)

        @pl.loop(0, rows_per_tile, step=BLK)
        def _(r):
            pltpu.async_copy(rows_b, acc_sh.at[pl.ds(r0 + r, BLK)], ssem_a)

        def run_half(half):
            pltpu.async_copy(src_hbm.at[wid, pl.ds(half * kh, kh)], src_v, gsem_a)
            pltpu.async_copy(dst_hbm.at[wid, pl.ds(half * kh, kh)], dst_v, gsem_b)
            pltpu.make_async_copy(
                src_hbm.at[wid, pl.ds(half * kh, kh)], src_v, gsem_a).wait()
            pltpu.make_async_copy(
                dst_hbm.at[wid, pl.ds(half * kh, kh)], dst_v, gsem_b).wait()

            pltpu.async_copy(g_hbm.at[src_v.at[0]], rows_a, gsem_a)

            if half == 0:
                @pl.loop(0, rows_per_tile, step=BLK)
                def _(r):
                    pltpu.make_async_copy(
                        rows_b, acc_sh.at[pl.ds(r0 + r, BLK)], ssem_a).wait()

                plsc.subcore_barrier()

            def pair(k, last):
                pltpu.async_copy(g_hbm.at[src_v.at[k + 1]], rows_b, gsem_b)
                pltpu.make_async_copy(g_hbm.at[src_v.at[k]], rows_a, gsem_a).wait()
                pltpu.async_copy(rows_a, acc_sh.at[dst_v.at[k]], ssem_a, add=True)
                pltpu.make_async_copy(g_hbm.at[src_v.at[k + 1]], rows_b, gsem_b).wait()
                pltpu.async_copy(rows_b, acc_sh.at[dst_v.at[k + 1]], ssem_b, add=True)
                pltpu.make_async_copy(rows_a, acc_sh.at[dst_v.at[k]], ssem_a).wait()

                if last is None:
                    pltpu.async_copy(g_hbm.at[src_v.at[k + 2]], rows_a, gsem_a)
                else:
                    @pl.when(last)
                    def _():
                        pltpu.async_copy(g_hbm.at[src_v.at[k + 2]], rows_a, gsem_a)

                pltpu.make_async_copy(rows_b, acc_sh.at[dst_v.at[k + 1]], ssem_b).wait()

            @pl.loop(0, kh // 4)
            def _(q):
                k = 4 * q
                pair(k, None)
                pair(k + 2, k + 4 < kh)

        run_half(0)
        run_half(1)
        plsc.subcore_barrier()
        pltpu.sync_copy(acc_sh.at[pl.ds(r0, rows_per_tile)],
                        out_hbm.at[c, pl.ds(r0, rows_per_tile)])

    return seg_kernel(g, src3, dst3, zeros_blk)


def _sc_degree(dst3, n_pad):
    kpt = dst3.shape[1]
    mesh = plsc.VectorSubcoreMesh(core_axis_name="c", subcore_axis_name="s")
    cp = pltpu.CompilerParams()
    if "needs_layout_passes" in pltpu.CompilerParams.__dataclass_fields__:
        cp = dataclasses.replace(cp, needs_layout_passes=False)

    @functools.partial(
        pl.kernel,
        out_type=jax.ShapeDtypeStruct((NW, n_pad), jnp.float32),
        mesh=mesh,
        compiler_params=cp,
        scratch_types=[
            pltpu.VMEM((kpt, BLK), jnp.int32),
            pltpu.VMEM((n_pad,), jnp.float32),
        ],
    )
    def deg_kernel(dst_hbm, out_hbm, dst_v, hist):
        c = lax.axis_index("c")
        s = lax.axis_index("s")
        wid = c * NS + s
        pltpu.sync_copy(dst_hbm.at[wid], dst_v)

        @pl.loop(0, n_pad, step=16)
        def _(i):
            hist[pl.ds(i, 16)] = jnp.zeros((16,), jnp.float32)

        ones = jnp.ones((16,), jnp.float32)

        @pl.loop(0, kpt)
        def _(j):
            @pl.loop(0, BLK, step=16)
            def _(k):
                idx = dst_v[j, pl.ds(k, 16)]
                plsc.addupdate_scatter(hist, [idx], ones)

        pltpu.sync_copy(hist, out_hbm.at[wid])

    return deg_kernel(dst3)


def _row_block(n):
    best = None
    for b in range(8, min(n, 2048) + 1, 8):
        if n % b == 0:
            best = b
    return best or n


def _mm_t(a, w):
    return lax.dot_general(a, w, (((1,), (1,)), ((), ())),
                           preferred_element_type=jnp.float32,
                           precision=lax.Precision.DEFAULT)


def _tc_lin_relu(x, W1, b1):
    def body(x_ref, w_ref, b_ref, o_ref):
        o_ref[...] = jnp.maximum(_mm_t(x_ref[...], w_ref[...]) + b_ref[...], 0.0)

    return pl.pallas_call(
        body,
        out_shape=jax.ShapeDtypeStruct((x.shape[0], W1.shape[0]), jnp.float32),
    )(x, W1, b1.reshape(1, -1))


def _tc_conv_pre(h, Wc, degp):
    n = h.shape[0]

    def body(h_ref, w_ref, d_ref, hp_ref, g_ref, dis_ref):
        hp = _mm_t(h_ref[...], w_ref[...])
        ones_nw = jnp.ones((NW, 1), jnp.float32)
        deg = lax.dot_general(d_ref[:, :n], ones_nw, (((0,), (0,)), ((), ())),
                              preferred_element_type=jnp.float32,
                              precision=lax.Precision.DEFAULT) + 1.0
        dis = lax.rsqrt(deg)
        hp_ref[...] = hp
        g_ref[...] = hp * dis
        dis_ref[...] = dis

    return pl.pallas_call(
        body,
        out_shape=(
            jax.ShapeDtypeStruct((n, Wc.shape[0]), jnp.float32),
            jax.ShapeDtypeStruct((n, Wc.shape[0]), jnp.float32),
            jax.ShapeDtypeStruct((n, 1), jnp.float32),
        ),
    )(h, Wc, degp)


def _tc_conv_post_pre(S, hp, dis, bc, Wc2):
    n = hp.shape[0]

    def body(s_ref, hp_ref, dis_ref, b_ref, w_ref, hp2_ref, g2_ref):
        dis = dis_ref[...]
        ssum = s_ref[0, :n] + s_ref[1, :n]
        agg = dis * ssum + dis * dis * hp_ref[...] + b_ref[...]
        x2 = jnp.maximum(agg, 0.0)
        hp2 = _mm_t(x2, w_ref[...])
        hp2_ref[...] = hp2
        g2_ref[...] = hp2 * dis

    return pl.pallas_call(
        body,
        out_shape=(
            jax.ShapeDtypeStruct((n, Wc2.shape[0]), jnp.float32),
            jax.ShapeDtypeStruct((n, Wc2.shape[0]), jnp.float32),
        ),
    )(S, hp, dis, bc.reshape(1, -1), Wc2)


def _tc_final(S, hp, dis, bc):
    n, dim = hp.shape

    def body(s_ref, hp_ref, dis_ref, b_ref, o_ref):
        dis = dis_ref[...]
        ssum = s_ref[0, :n] + s_ref[1, :n]
        z = dis * ssum + dis * dis * hp_ref[...] + b_ref[...]
        m = jnp.max(z, axis=1, keepdims=True)
        zs = z - m
        lse = jnp.log(jnp.sum(jnp.exp(zs), axis=1, keepdims=True))
        o_ref[...] = zs - lse

    return pl.pallas_call(
        body,
        out_shape=jax.ShapeDtypeStruct((n, dim), jnp.float32),
    )(S, hp, dis, bc.reshape(1, -1))


def kernel(x, edge_index, W1, b1, Wc1, bc1, Wc2, bc2):
    n, _ = x.shape
    e = edge_index.shape[1]
    dh = W1.shape[0]

    n_pad = _round_up(n + 128, NS * BLK)
    kpt = _round_up(-(-e // (NW * BLK)), 8)
    e_pad = NW * kpt * BLK

    pad_iota = np.arange(e_pad - e, dtype=np.int32)
    pad_src = jnp.asarray(pad_iota % n)
    pad_dst = jnp.asarray(n + pad_iota % (n_pad - n))
    src = jnp.concatenate([edge_index[0], pad_src])
    dst = jnp.concatenate([edge_index[1], pad_dst])
    src3 = src.reshape(NW, kpt, BLK)
    dst3 = dst.reshape(NW, kpt, BLK)

    zeros_blk = jnp.zeros((BLK, dh), jnp.float32)

    degp = _sc_degree(dst3, n_pad)
    h1 = _tc_lin_relu(x, W1, b1)
    hp1, g1, dis = _tc_conv_pre(h1, Wc1, degp)
    S1 = _sc_segsum(g1, src3, dst3, zeros_blk, n_pad)
    hp2, g2 = _tc_conv_post_pre(S1, hp1, dis, bc1, Wc2)
    S2 = _sc_segsum(g2, src3, dst3, zeros_blk, n_pad)
    return _tc_final(S2, hp2, dis, bc2)
